# baseline (speedup 1.0000x reference)
import functools

import jax
import jax.numpy as jnp
from jax import lax
from jax.experimental import pallas as pl
from jax.experimental.pallas import tpu as pltpu
from jax.experimental.pallas import tpu_sc as plsc

B, L = 4096, 200
N = B * L
POI_DIM = 128
HOUR_DIM = 16
WKND_DIM = 8
USER_DIM = 64
HW_DIM = HOUR_DIM + WKND_DIM
TAIL_DIM = HW_DIM + USER_DIM
OUT_DIM = POI_DIM + TAIL_DIM

NUM_CORES = 2
NUM_SUBCORES = 16
NW = NUM_CORES * NUM_SUBCORES
ROWS_PER_W = B // NW
CHUNK = L
NCHUNK = ROWS_PER_W
IDX_GROUPS = ((0, 128), (128, 72))
HW_OFFS = tuple([k * 16 for k in range(L // 16)] + [L - 16])


def _emb_body(poi_hbm, hour_hbm, wknd_hbm, user_hbm,
              hw_tbl, u_tbl, out_hbm,
              h_idx, w_idx, u_idx, hw_idx, hw_rows, t_rows, p_rows,
              sem_i0, sem_i1, sem_g0, sem_g1, sem_w0, sem_w1):
    wid = lax.axis_index("s") * NUM_CORES + lax.axis_index("c")
    w_base = wid * ROWS_PER_W
    sem_i = (sem_i0, sem_i1)
    sem_g = (sem_g0, sem_g1)
    sem_w = (sem_w0, sem_w1)

    def fire_idx(i, p):
        tsl = pl.ds((w_base + i) * CHUNK, CHUNK)
        pltpu.async_copy(hour_hbm.at[tsl], h_idx.at[p], sem_i[p])
        pltpu.async_copy(wknd_hbm.at[tsl], w_idx.at[p], sem_i[p])
        pltpu.async_copy(user_hbm.at[tsl], u_idx.at[p], sem_i[p])

    def wait_idx(p):
        for dst in (h_idx, w_idx, u_idx):
            pltpu.make_async_copy(hour_hbm.at[pl.ds(0, CHUNK)],
                                  dst.at[p], sem_i[p]).wait()

    def compute_hw(p):
        for off in HW_OFFS:
            sl = pl.ds(off, 16)
            hw_idx[p, sl] = h_idx[p, sl] * 3 + w_idx[p, sl]

    def fire_pg(i, p):
        for (off, ln) in IDX_GROUPS:
            d = pl.ds(off, ln)
            pltpu.async_copy(u_tbl.at[u_idx.at[p, d]], t_rows.at[p, d],
                             sem_g[p])
            pltpu.async_copy(hw_tbl.at[hw_idx.at[p, d]], hw_rows.at[p, d],
                             sem_g[p])
        pltpu.async_copy(poi_hbm.at[w_base + i], p_rows.at[p], sem_g[p])

    def wait_pg(p):
        for (off, ln) in IDX_GROUPS:
            d = pl.ds(off, ln)
            pltpu.make_async_copy(u_tbl.at[pl.ds(0, ln)],
                                  t_rows.at[p, d], sem_g[p]).wait()
            pltpu.make_async_copy(u_tbl.at[pl.ds(0, ln), pl.ds(0, HW_DIM)],
                                  hw_rows.at[p, d], sem_g[p]).wait()
        pltpu.make_async_copy(poi_hbm.at[0], p_rows.at[p], sem_g[p]).wait()

    def overlay(p):
        def tok_body(t):
            t_rows[p, t, pl.ds(0, 16)] = hw_rows[p, t, pl.ds(0, 16)]
            t_rows[p, t, pl.ds(8, 16)] = hw_rows[p, t, pl.ds(8, 16)]
        pl.loop(0, CHUNK, unroll=8)(tok_body)

    def fire_write(i, p):
        out = out_hbm.at[pl.ds((w_base + i) * CHUNK, CHUNK)]
        pltpu.async_copy(p_rows.at[p], out.at[:, pl.ds(0, POI_DIM)],
                         sem_w[p])
        pltpu.async_copy(t_rows.at[p], out.at[:, pl.ds(POI_DIM, TAIL_DIM)],
                         sem_w[p])

    def drain_write(p):
        out = out_hbm.at[pl.ds(0, CHUNK)]
        pltpu.make_async_copy(p_rows.at[p], out.at[:, pl.ds(0, POI_DIM)],
                              sem_w[p]).wait()
        pltpu.make_async_copy(t_rows.at[p],
                              out.at[:, pl.ds(POI_DIM, TAIL_DIM)],
                              sem_w[p]).wait()

    fire_idx(0, 0)
    wait_idx(0)
    compute_hw(0)
    fire_pg(0, 0)
    fire_idx(1, 1)

    def step(i, p):
        @pl.when(i + 1 < NCHUNK)
        def _():
            wait_idx(1 - p)
            compute_hw(1 - p)
        wait_pg(p)
        @pl.when(i + 1 < NCHUNK)
        def _():
            fire_pg(i + 1, 1 - p)
        @pl.when(i + 2 < NCHUNK)
        def _():
            fire_idx(i + 2, p)
        @pl.when(i >= 2)
        def _():
            drain_write(p)
        overlay(p)
        fire_write(i, p)

    def pair_body(j):
        step(2 * j, 0)
        step(2 * j + 1, 1)
    pl.loop(0, NCHUNK // 2)(pair_body)
    drain_write(0)
    drain_write(1)


_mesh = plsc.VectorSubcoreMesh(core_axis_name="c", subcore_axis_name="s")

_emb_kernel = functools.partial(
    pl.kernel,
    out_type=jax.ShapeDtypeStruct((N, OUT_DIM), jnp.float32),
    mesh=_mesh,
    compiler_params=pltpu.CompilerParams(use_tc_tiling_on_sc=False),
    scratch_types=[
        pltpu.VMEM((2, CHUNK), jnp.int32),
        pltpu.VMEM((2, CHUNK), jnp.int32),
        pltpu.VMEM((2, CHUNK), jnp.int32),
        pltpu.VMEM((2, CHUNK), jnp.int32),
        pltpu.VMEM((2, CHUNK, HW_DIM), jnp.float32),
        pltpu.VMEM((2, CHUNK, TAIL_DIM), jnp.float32),
        pltpu.VMEM((2, CHUNK, POI_DIM), jnp.float32),
        pltpu.SemaphoreType.DMA,
        pltpu.SemaphoreType.DMA,
        pltpu.SemaphoreType.DMA,
        pltpu.SemaphoreType.DMA,
        pltpu.SemaphoreType.DMA,
        pltpu.SemaphoreType.DMA,
    ],
)(_emb_body)


@jax.jit
def kernel(seq_poi_embeddings, hour_set, isweekend_set, user_set,
           hour_table, isweekend_table, user_table):
    hour = hour_set.reshape(N)
    wknd = isweekend_set.reshape(N)
    user = user_set.reshape(N)
    h_tbl = hour_table.at[0].set(0.0)
    w_tbl = isweekend_table.at[0].set(0.0)
    hw_tbl = jnp.concatenate(
        [jnp.broadcast_to(h_tbl[:, None, :], (25, 3, HOUR_DIM)),
         jnp.broadcast_to(w_tbl[None, :, :], (25, 3, WKND_DIM))],
        axis=2).reshape(75, HW_DIM)
    u_tbl = jnp.concatenate(
        [jnp.zeros((100001, HW_DIM), jnp.float32),
         user_table.at[0].set(0.0)], axis=1)
    out = _emb_kernel(seq_poi_embeddings, hour, wknd, user, hw_tbl, u_tbl)
    return out.reshape(B, L, OUT_DIM)
